# baseline (speedup 1.0000x reference)
import functools

import jax
import jax.numpy as jnp
from jax import lax
from jax.experimental import pallas as pl
from jax.experimental.pallas import tpu as pltpu
from jax.experimental.pallas import tpu_sc as plsc

N, E, D, H, O = 10000, 160000, 256, 256, 128
NC, NS = 2, 16
NH = N // 2
CAP = 81920
EPT = CAP // NS
CH = 64
NCHUNK = EPT // CH
GSZ = 4
NGROUP = NCHUNK // GSZ
NPAD = 10240
ACCR = 5120
DUMMY = ACCR - 1
RPT = ACCR // NS
GBN = 1000
NBH = NH // GBN

_sc_mesh = plsc.VectorSubcoreMesh(core_axis_name="c", subcore_axis_name="s")


@functools.partial(
    pl.kernel,
    out_type=jax.ShapeDtypeStruct((NC, ACCR, 2, 128), jnp.float32),
    mesh=_sc_mesh,
    scratch_types=[
        pltpu.VMEM((2 * GSZ, CH), jnp.int32),
        pltpu.VMEM((2 * GSZ, CH), jnp.int32),
        pltpu.VMEM((CH, 2, 128), jnp.float32),
        pltpu.VMEM((CH, 2, 128), jnp.float32),
        pltpu.VMEM_SHARED((ACCR, 2, 128), jnp.float32),
        pltpu.SemaphoreType.DMA,
        pltpu.SemaphoreType.DMA,
        pltpu.SemaphoreType.DMA,
        pltpu.SemaphoreType.DMA,
    ],
)
def _seg_sum(xf, idxg, zeros, out, ig0, ig1, rows0, rows1, acc,
             si0, si1, sg0, sg1):
    c = lax.axis_index("c")
    s = lax.axis_index("s")
    tidx = idxg.at[c].at[s]
    pltpu.sync_copy(zeros, acc.at[pl.ds(s * RPT, RPT)])
    pltpu.sync_copy(tidx.at[0], ig0)
    plsc.subcore_barrier()
    pltpu.async_copy(xf.at[ig0.at[0]], rows0, sg0)

    NG2 = NGROUP // 2

    def body(g2, carry):
        for gg in range(2):
            g = 2 * g2 + gg
            ig, ign = (ig0, ig1) if gg == 0 else (ig1, ig0)
            si_n = si1 if gg == 0 else si0
            for c4 in range(GSZ):
                rb, rn = (rows0, rows1) if c4 % 2 == 0 else (rows1, rows0)
                sb, sn = (sg0, sg1) if c4 % 2 == 0 else (sg1, sg0)
                last_chunk = gg == 1 and c4 == GSZ - 1

                if c4 == 1:
                    if gg == 0:
                        pltpu.async_copy(tidx.at[g + 1], ign, si_n)
                    else:
                        @pl.when(g2 < NG2 - 1)
                        def _(ign=ign, si_n=si_n, g=g):
                            pltpu.async_copy(tidx.at[g + 1], ign, si_n)

                if c4 < GSZ - 1:
                    pltpu.async_copy(xf.at[ig.at[c4 + 1]], rn, sn)
                elif not last_chunk:
                    pltpu.make_async_copy(tidx.at[g + 1], ign, si_n).wait()
                    pltpu.async_copy(xf.at[ign.at[0]], rn, sn)
                else:
                    @pl.when(g2 < NG2 - 1)
                    def _(ign=ign, si_n=si_n, rn=rn, sn=sn, g=g):
                        pltpu.make_async_copy(tidx.at[g + 1], ign, si_n).wait()
                        pltpu.async_copy(xf.at[ign.at[0]], rn, sn)

                pltpu.make_async_copy(xf.at[ig.at[c4]], rb, sb).wait()
                pltpu.sync_copy(rb, acc.at[ig.at[GSZ + c4]], add=True)
        return carry

    lax.fori_loop(0, NG2, body, 0)
    plsc.subcore_barrier()
    pltpu.sync_copy(acc.at[pl.ds(s * RPT, RPT)],
                    out.at[c].at[pl.ds(s * RPT, RPT)])


def _lrelu(x):
    return jnp.where(x >= 0, x, 0.2 * x)


def _pick_half(i, agg_ref):
    return jnp.where(i < NBH, agg_ref[0], agg_ref[1])


def _dense0_body(agg_ref, x_ref, wn_ref, ws_ref, b_ref, h_ref, csx_ref,
                 csh_ref):
    i = pl.program_id(0)
    agg = _pick_half(i, agg_ref)
    x = x_ref[...]
    h = jnp.dot(agg, wn_ref[...], preferred_element_type=jnp.float32)
    h += jnp.dot(x, ws_ref[...], preferred_element_type=jnp.float32)
    h += b_ref[...]
    h = _lrelu(_lrelu(h))
    h_ref[...] = h

    @pl.when(i == 0)
    def _():
        csx_ref[...] = jnp.zeros_like(csx_ref)
        csh_ref[...] = jnp.zeros_like(csh_ref)

    csx_ref[...] += jnp.sum(x, axis=0, keepdims=True)
    csh_ref[...] += jnp.sum(h, axis=0, keepdims=True)


def _dense1_body(agg_ref, x_ref, wn_ref, ws_ref, b_ref, cs0_ref, cs1_ref,
                 r0_ref, r1_ref, r2_ref, rb0_ref, rb1_ref, rb2_ref, out_ref):
    i = pl.program_id(0)
    agg = _pick_half(i, agg_ref)
    x = x_ref[...]
    h = jnp.dot(agg, wn_ref[...], preferred_element_type=jnp.float32)
    h += jnp.dot(x, ws_ref[...], preferred_element_type=jnp.float32)
    h += b_ref[...]
    h = _lrelu(_lrelu(h))
    csh = jnp.sum(h, axis=0, keepdims=True)

    @pl.when(i == 0)
    def _():
        out_ref[...] = (
            jnp.dot(cs0_ref[...], r0_ref[...], preferred_element_type=jnp.float32)
            + jnp.dot(cs1_ref[...], r1_ref[...], preferred_element_type=jnp.float32)
            + float(N) * (rb0_ref[...] + rb1_ref[...] + rb2_ref[...])
        )

    out_ref[...] += jnp.dot(csh, r2_ref[...], preferred_element_type=jnp.float32)


def _agg_spec():
    return pl.BlockSpec((NC, GBN, D), lambda i: (0, i % NBH, 0))


_dense0 = pl.pallas_call(
    _dense0_body,
    grid=(N // GBN,),
    in_specs=[
        _agg_spec(),
        pl.BlockSpec((GBN, D), lambda i: (i, 0)),
        pl.BlockSpec((D, H), lambda i: (0, 0)),
        pl.BlockSpec((D, H), lambda i: (0, 0)),
        pl.BlockSpec((1, H), lambda i: (0, 0)),
    ],
    out_specs=[
        pl.BlockSpec((GBN, H), lambda i: (i, 0)),
        pl.BlockSpec((1, D), lambda i: (0, 0)),
        pl.BlockSpec((1, H), lambda i: (0, 0)),
    ],
    out_shape=[
        jax.ShapeDtypeStruct((NPAD, H), jnp.float32),
        jax.ShapeDtypeStruct((1, D), jnp.float32),
        jax.ShapeDtypeStruct((1, H), jnp.float32),
    ],
)

_dense1 = pl.pallas_call(
    _dense1_body,
    grid=(N // GBN,),
    in_specs=[
        _agg_spec(),
        pl.BlockSpec((GBN, D), lambda i: (i, 0)),
        pl.BlockSpec((H, H), lambda i: (0, 0)),
        pl.BlockSpec((H, H), lambda i: (0, 0)),
        pl.BlockSpec((1, H), lambda i: (0, 0)),
        pl.BlockSpec((1, D), lambda i: (0, 0)),
        pl.BlockSpec((1, H), lambda i: (0, 0)),
        pl.BlockSpec((D, O), lambda i: (0, 0)),
        pl.BlockSpec((H, O), lambda i: (0, 0)),
        pl.BlockSpec((H, O), lambda i: (0, 0)),
        pl.BlockSpec((1, O), lambda i: (0, 0)),
        pl.BlockSpec((1, O), lambda i: (0, 0)),
        pl.BlockSpec((1, O), lambda i: (0, 0)),
    ],
    out_specs=pl.BlockSpec((1, O), lambda i: (0, 0)),
    out_shape=jax.ShapeDtypeStruct((1, O), jnp.float32),
)


def kernel(nfeats, efeats, edge_index, Wself0, Wneigh0, b0, Wself1, Wneigh1,
           b1, R0, Rb0, R1, Rb1, R2, Rb2):
    src = edge_index[0]
    dst = edge_index[1]
    perm = jnp.argsort((dst >= NH).astype(jnp.int32))
    srcs = src[perm]
    dsts = dst[perm]

    def half(c):
        sl = slice(0, CAP) if c == 0 else slice(E - CAP, E)
        sc_, dc_ = srcs[sl], dsts[sl]
        ok = (dc_ >= c * NH) & (dc_ < (c + 1) * NH)
        s_ = jnp.where(ok, sc_, NPAD - 1).reshape(NS, NGROUP, GSZ, CH)
        d_ = jnp.where(ok, dc_ - c * NH, DUMMY).reshape(NS, NGROUP, GSZ, CH)
        return jnp.concatenate([s_, d_], axis=2)

    idxg = jnp.stack([half(0), half(1)])
    zeros = jnp.zeros((RPT, 2, 128), jnp.float32)

    x0f = jnp.pad(nfeats, ((0, NPAD - N), (0, 0))).reshape(NPAD, 2, 128)
    agg0 = _seg_sum(x0f, idxg, zeros).reshape(NC, ACCR, D)
    h1, cs0, cs1 = _dense0(agg0, nfeats, Wneigh0, Wself0, b0.reshape(1, H))
    agg1 = _seg_sum(h1.reshape(NPAD, 2, 128), idxg, zeros).reshape(NC, ACCR, D)
    out = _dense1(agg1, h1, Wneigh1, Wself1, b1.reshape(1, H), cs0, cs1,
                  R0, R1, R2, Rb0.reshape(1, O), Rb1.reshape(1, O),
                  Rb2.reshape(1, O))
    return out


# speedup vs baseline: 1.6833x; 1.6833x over previous
import functools

import jax
import jax.numpy as jnp
from jax import lax
from jax.experimental import pallas as pl
from jax.experimental.pallas import tpu as pltpu
from jax.experimental.pallas import tpu_sc as plsc

N, E, D, H, O = 10000, 160000, 256, 256, 128
HALF = 128
NC, NS = 2, 16
EPT = E // NS
CH = 80
NCHUNK = 125
NPAD = 10240
RPT = NPAD // NS

_sc_mesh = plsc.VectorSubcoreMesh(core_axis_name="c", subcore_axis_name="s")


@functools.partial(
    pl.kernel,
    out_type=jax.ShapeDtypeStruct((NC, NPAD, HALF), jnp.float32),
    mesh=_sc_mesh,
    scratch_types=[
        pltpu.VMEM((NCHUNK, CH), jnp.int32),
        pltpu.VMEM((NCHUNK, CH), jnp.int32),
        pltpu.VMEM((CH, HALF), jnp.float32),
        pltpu.VMEM_SHARED((NPAD, HALF), jnp.float32),
        pltpu.SemaphoreType.DMA,
    ],
)
def _seg_sum(xh, srcr, dstr, zeros, out, idx_s, idx_d, rows, acc, sem):
    c = lax.axis_index("c")
    s = lax.axis_index("s")
    pltpu.sync_copy(zeros, acc.at[pl.ds(s * RPT, RPT)])
    pltpu.sync_copy(srcr.at[c].at[s], idx_s)
    pltpu.sync_copy(dstr.at[s], idx_d)
    plsc.subcore_barrier()

    def chunk(j, carry):
        pltpu.async_copy(xh.at[idx_s.at[j]], rows, sem).wait()
        pltpu.sync_copy(rows, acc.at[idx_d.at[j]], add=True)
        return carry

    lax.fori_loop(0, NCHUNK, chunk, 0)
    plsc.subcore_barrier()
    pltpu.sync_copy(acc.at[pl.ds(s * RPT, RPT)], out.at[c].at[pl.ds(s * RPT, RPT)])


def _lrelu(x):
    return jnp.where(x >= 0, x, 0.2 * x)


def _dense0_body(agg_ref, x_ref, wn_ref, ws_ref, b_ref, h_ref, csx_ref, csh_ref):
    i = pl.program_id(0)
    agg = jnp.concatenate([agg_ref[0], agg_ref[1]], axis=1)
    x = x_ref[...]
    h = jnp.dot(agg, wn_ref[...], preferred_element_type=jnp.float32)
    h += jnp.dot(x, ws_ref[...], preferred_element_type=jnp.float32)
    h += b_ref[...]
    h = _lrelu(_lrelu(h))
    h_ref[0] = h[:, :HALF]
    h_ref[1] = h[:, HALF:]

    @pl.when(i == 0)
    def _():
        csx_ref[...] = jnp.zeros_like(csx_ref)
        csh_ref[...] = jnp.zeros_like(csh_ref)

    csx_ref[...] += jnp.sum(x, axis=0, keepdims=True)
    csh_ref[...] += jnp.sum(h, axis=0, keepdims=True)


def _dense1_body(agg_ref, x_ref, wn_ref, ws_ref, b_ref, cs0_ref, cs1_ref,
                 r0_ref, r1_ref, r2_ref, rb0_ref, rb1_ref, rb2_ref, out_ref):
    i = pl.program_id(0)
    agg = jnp.concatenate([agg_ref[0], agg_ref[1]], axis=1)
    x = jnp.concatenate([x_ref[0], x_ref[1]], axis=1)
    h = jnp.dot(agg, wn_ref[...], preferred_element_type=jnp.float32)
    h += jnp.dot(x, ws_ref[...], preferred_element_type=jnp.float32)
    h += b_ref[...]
    h = _lrelu(_lrelu(h))
    csh = jnp.sum(h, axis=0, keepdims=True)

    @pl.when(i == 0)
    def _():
        out_ref[...] = (
            jnp.dot(cs0_ref[...], r0_ref[...], preferred_element_type=jnp.float32)
            + jnp.dot(cs1_ref[...], r1_ref[...], preferred_element_type=jnp.float32)
            + float(N) * (rb0_ref[...] + rb1_ref[...] + rb2_ref[...])
        )

    out_ref[...] += jnp.dot(csh, r2_ref[...], preferred_element_type=jnp.float32)


GBN = 1000

_dense0 = pl.pallas_call(
    _dense0_body,
    grid=(N // GBN,),
    in_specs=[
        pl.BlockSpec((NC, GBN, HALF), lambda i: (0, i, 0)),
        pl.BlockSpec((GBN, D), lambda i: (i, 0)),
        pl.BlockSpec((D, H), lambda i: (0, 0)),
        pl.BlockSpec((D, H), lambda i: (0, 0)),
        pl.BlockSpec((1, H), lambda i: (0, 0)),
    ],
    out_specs=[
        pl.BlockSpec((NC, GBN, HALF), lambda i: (0, i, 0)),
        pl.BlockSpec((1, D), lambda i: (0, 0)),
        pl.BlockSpec((1, H), lambda i: (0, 0)),
    ],
    out_shape=[
        jax.ShapeDtypeStruct((NC, NPAD, HALF), jnp.float32),
        jax.ShapeDtypeStruct((1, D), jnp.float32),
        jax.ShapeDtypeStruct((1, H), jnp.float32),
    ],
)

_dense1 = pl.pallas_call(
    _dense1_body,
    grid=(N // GBN,),
    in_specs=[
        pl.BlockSpec((NC, GBN, HALF), lambda i: (0, i, 0)),
        pl.BlockSpec((NC, GBN, HALF), lambda i: (0, i, 0)),
        pl.BlockSpec((H, H), lambda i: (0, 0)),
        pl.BlockSpec((H, H), lambda i: (0, 0)),
        pl.BlockSpec((1, H), lambda i: (0, 0)),
        pl.BlockSpec((1, D), lambda i: (0, 0)),
        pl.BlockSpec((1, H), lambda i: (0, 0)),
        pl.BlockSpec((D, O), lambda i: (0, 0)),
        pl.BlockSpec((H, O), lambda i: (0, 0)),
        pl.BlockSpec((H, O), lambda i: (0, 0)),
        pl.BlockSpec((1, O), lambda i: (0, 0)),
        pl.BlockSpec((1, O), lambda i: (0, 0)),
        pl.BlockSpec((1, O), lambda i: (0, 0)),
    ],
    out_specs=pl.BlockSpec((1, O), lambda i: (0, 0)),
    out_shape=jax.ShapeDtypeStruct((1, O), jnp.float32),
)


def kernel(nfeats, efeats, edge_index, Wself0, Wneigh0, b0, Wself1, Wneigh1,
           b1, R0, Rb0, R1, Rb1, R2, Rb2):
    src = edge_index[0]
    dst = edge_index[1]
    src_adj = jnp.stack([src, src + NPAD]).reshape(NC, NS, NCHUNK, CH)
    dstr = dst.reshape(NS, NCHUNK, CH)
    zeros = jnp.zeros((RPT, HALF), jnp.float32)

    x0h = jnp.concatenate(
        [nfeats[:, :HALF], jnp.zeros((NPAD - N, HALF), jnp.float32),
         nfeats[:, HALF:]], axis=0)
    x0h = jnp.concatenate([x0h, jnp.zeros((NPAD - N, HALF), jnp.float32)], axis=0)
    agg0 = _seg_sum(x0h, src_adj, dstr, zeros)
    h1, cs0, cs1 = _dense0(agg0, nfeats, Wneigh0, Wself0, b0.reshape(1, H))
    agg1 = _seg_sum(h1.reshape(NC * NPAD, HALF), src_adj, dstr, zeros)
    out = _dense1(agg1, h1, Wneigh1, Wself1, b1.reshape(1, H), cs0, cs1,
                  R0, R1, R2, Rb0.reshape(1, O), Rb1.reshape(1, O),
                  Rb2.reshape(1, O))
    return out


# speedup vs baseline: 1.7971x; 1.0676x over previous
import functools

import jax
import jax.numpy as jnp
from jax import lax
from jax.experimental import pallas as pl
from jax.experimental.pallas import tpu as pltpu
from jax.experimental.pallas import tpu_sc as plsc

N, E, D, H, O = 10000, 160000, 256, 256, 128
HALF = 128
NC, NS = 2, 16
EPT = E // NS
CH = 100
NCHUNK = 100
NPAD = 10240
RPT = NPAD // NS

_sc_mesh = plsc.VectorSubcoreMesh(core_axis_name="c", subcore_axis_name="s")


@functools.partial(
    pl.kernel,
    out_type=jax.ShapeDtypeStruct((NC, NPAD, HALF), jnp.float32),
    mesh=_sc_mesh,
    scratch_types=[
        pltpu.VMEM((NCHUNK, CH), jnp.int32),
        pltpu.VMEM((NCHUNK, CH), jnp.int32),
        pltpu.VMEM((CH, HALF), jnp.float32),
        pltpu.VMEM_SHARED((NPAD, HALF), jnp.float32),
        pltpu.SemaphoreType.DMA,
    ],
)
def _seg_sum(xh, srcr, dstr, zeros, out, idx_s, idx_d, rows, acc, sem):
    c = lax.axis_index("c")
    s = lax.axis_index("s")
    pltpu.sync_copy(zeros, acc.at[pl.ds(s * RPT, RPT)])
    pltpu.sync_copy(srcr.at[c].at[s], idx_s)
    pltpu.sync_copy(dstr.at[s], idx_d)
    plsc.subcore_barrier()

    def chunk(j, carry):
        pltpu.async_copy(xh.at[idx_s.at[j]], rows, sem).wait()
        pltpu.sync_copy(rows, acc.at[idx_d.at[j]], add=True)
        return carry

    lax.fori_loop(0, NCHUNK, chunk, 0)
    plsc.subcore_barrier()
    pltpu.sync_copy(acc.at[pl.ds(s * RPT, RPT)], out.at[c].at[pl.ds(s * RPT, RPT)])


def _lrelu(x):
    return jnp.where(x >= 0, x, 0.2 * x)


def _dense0_body(agg_ref, x_ref, wn_ref, ws_ref, b_ref, h_ref, csx_ref, csh_ref):
    i = pl.program_id(0)
    agg = jnp.concatenate([agg_ref[0], agg_ref[1]], axis=1)
    x = x_ref[...]
    h = jnp.dot(agg, wn_ref[...], preferred_element_type=jnp.float32)
    h += jnp.dot(x, ws_ref[...], preferred_element_type=jnp.float32)
    h += b_ref[...]
    h = _lrelu(_lrelu(h))
    h_ref[0] = h[:, :HALF]
    h_ref[1] = h[:, HALF:]

    @pl.when(i == 0)
    def _():
        csx_ref[...] = jnp.zeros_like(csx_ref)
        csh_ref[...] = jnp.zeros_like(csh_ref)

    csx_ref[...] += jnp.sum(x, axis=0, keepdims=True)
    csh_ref[...] += jnp.sum(h, axis=0, keepdims=True)


def _dense1_body(agg_ref, x_ref, wn_ref, ws_ref, b_ref, cs0_ref, cs1_ref,
                 r0_ref, r1_ref, r2_ref, rb0_ref, rb1_ref, rb2_ref, out_ref):
    i = pl.program_id(0)
    agg = jnp.concatenate([agg_ref[0], agg_ref[1]], axis=1)
    x = jnp.concatenate([x_ref[0], x_ref[1]], axis=1)
    h = jnp.dot(agg, wn_ref[...], preferred_element_type=jnp.float32)
    h += jnp.dot(x, ws_ref[...], preferred_element_type=jnp.float32)
    h += b_ref[...]
    h = _lrelu(_lrelu(h))
    csh = jnp.sum(h, axis=0, keepdims=True)

    @pl.when(i == 0)
    def _():
        out_ref[...] = (
            jnp.dot(cs0_ref[...], r0_ref[...], preferred_element_type=jnp.float32)
            + jnp.dot(cs1_ref[...], r1_ref[...], preferred_element_type=jnp.float32)
            + float(N) * (rb0_ref[...] + rb1_ref[...] + rb2_ref[...])
        )

    out_ref[...] += jnp.dot(csh, r2_ref[...], preferred_element_type=jnp.float32)


GBN = 1000

_dense0 = pl.pallas_call(
    _dense0_body,
    grid=(N // GBN,),
    in_specs=[
        pl.BlockSpec((NC, GBN, HALF), lambda i: (0, i, 0)),
        pl.BlockSpec((GBN, D), lambda i: (i, 0)),
        pl.BlockSpec((D, H), lambda i: (0, 0)),
        pl.BlockSpec((D, H), lambda i: (0, 0)),
        pl.BlockSpec((1, H), lambda i: (0, 0)),
    ],
    out_specs=[
        pl.BlockSpec((NC, GBN, HALF), lambda i: (0, i, 0)),
        pl.BlockSpec((1, D), lambda i: (0, 0)),
        pl.BlockSpec((1, H), lambda i: (0, 0)),
    ],
    out_shape=[
        jax.ShapeDtypeStruct((NC, NPAD, HALF), jnp.float32),
        jax.ShapeDtypeStruct((1, D), jnp.float32),
        jax.ShapeDtypeStruct((1, H), jnp.float32),
    ],
)

_dense1 = pl.pallas_call(
    _dense1_body,
    grid=(N // GBN,),
    in_specs=[
        pl.BlockSpec((NC, GBN, HALF), lambda i: (0, i, 0)),
        pl.BlockSpec((NC, GBN, HALF), lambda i: (0, i, 0)),
        pl.BlockSpec((H, H), lambda i: (0, 0)),
        pl.BlockSpec((H, H), lambda i: (0, 0)),
        pl.BlockSpec((1, H), lambda i: (0, 0)),
        pl.BlockSpec((1, D), lambda i: (0, 0)),
        pl.BlockSpec((1, H), lambda i: (0, 0)),
        pl.BlockSpec((D, O), lambda i: (0, 0)),
        pl.BlockSpec((H, O), lambda i: (0, 0)),
        pl.BlockSpec((H, O), lambda i: (0, 0)),
        pl.BlockSpec((1, O), lambda i: (0, 0)),
        pl.BlockSpec((1, O), lambda i: (0, 0)),
        pl.BlockSpec((1, O), lambda i: (0, 0)),
    ],
    out_specs=pl.BlockSpec((1, O), lambda i: (0, 0)),
    out_shape=jax.ShapeDtypeStruct((1, O), jnp.float32),
)


def kernel(nfeats, efeats, edge_index, Wself0, Wneigh0, b0, Wself1, Wneigh1,
           b1, R0, Rb0, R1, Rb1, R2, Rb2):
    src = edge_index[0]
    dst = edge_index[1]
    src_adj = jnp.stack([src, src + NPAD]).reshape(NC, NS, NCHUNK, CH)
    dstr = dst.reshape(NS, NCHUNK, CH)
    zeros = jnp.zeros((RPT, HALF), jnp.float32)

    x0h = jnp.concatenate(
        [nfeats[:, :HALF], jnp.zeros((NPAD - N, HALF), jnp.float32),
         nfeats[:, HALF:]], axis=0)
    x0h = jnp.concatenate([x0h, jnp.zeros((NPAD - N, HALF), jnp.float32)], axis=0)
    agg0 = _seg_sum(x0h, src_adj, dstr, zeros)
    h1, cs0, cs1 = _dense0(agg0, nfeats, Wneigh0, Wself0, b0.reshape(1, H))
    agg1 = _seg_sum(h1.reshape(NC * NPAD, HALF), src_adj, dstr, zeros)
    out = _dense1(agg1, h1, Wneigh1, Wself1, b1.reshape(1, H), cs0, cs1,
                  R0, R1, R2, Rb0.reshape(1, O), Rb1.reshape(1, O),
                  Rb2.reshape(1, O))
    return out


# speedup vs baseline: 1.9132x; 1.0646x over previous
import functools

import jax
import jax.numpy as jnp
from jax import lax
from jax.experimental import pallas as pl
from jax.experimental.pallas import tpu as pltpu
from jax.experimental.pallas import tpu_sc as plsc

N, E, D, H, O = 10000, 160000, 256, 256, 128
HALF = 128
NC, NS = 2, 16
EPT = E // NS
CH = 125
NCHUNK = 80
NPAD = 10240
RPT = NPAD // NS

_sc_mesh = plsc.VectorSubcoreMesh(core_axis_name="c", subcore_axis_name="s")


@functools.partial(
    pl.kernel,
    out_type=jax.ShapeDtypeStruct((NC, NPAD, HALF), jnp.float32),
    mesh=_sc_mesh,
    scratch_types=[
        pltpu.VMEM((NCHUNK, CH), jnp.int32),
        pltpu.VMEM((NCHUNK, CH), jnp.int32),
        pltpu.VMEM((CH, HALF), jnp.float32),
        pltpu.VMEM_SHARED((NPAD, HALF), jnp.float32),
        pltpu.SemaphoreType.DMA,
    ],
)
def _seg_sum(xh, srcr, dstr, zeros, out, idx_s, idx_d, rows, acc, sem):
    c = lax.axis_index("c")
    s = lax.axis_index("s")
    pltpu.sync_copy(zeros, acc.at[pl.ds(s * RPT, RPT)])
    pltpu.sync_copy(srcr.at[c].at[s], idx_s)
    pltpu.sync_copy(dstr.at[s], idx_d)
    plsc.subcore_barrier()

    def chunk(j, carry):
        pltpu.async_copy(xh.at[idx_s.at[j]], rows, sem).wait()
        pltpu.sync_copy(rows, acc.at[idx_d.at[j]], add=True)
        return carry

    lax.fori_loop(0, NCHUNK, chunk, 0)
    plsc.subcore_barrier()
    pltpu.sync_copy(acc.at[pl.ds(s * RPT, RPT)], out.at[c].at[pl.ds(s * RPT, RPT)])


def _lrelu(x):
    return jnp.where(x >= 0, x, 0.2 * x)


def _dense0_body(agg_ref, x_ref, wn_ref, ws_ref, b_ref, h_ref, csx_ref, csh_ref):
    i = pl.program_id(0)
    agg = jnp.concatenate([agg_ref[0], agg_ref[1]], axis=1)
    x = x_ref[...]
    h = jnp.dot(agg, wn_ref[...], preferred_element_type=jnp.float32)
    h += jnp.dot(x, ws_ref[...], preferred_element_type=jnp.float32)
    h += b_ref[...]
    h = _lrelu(_lrelu(h))
    h_ref[0] = h[:, :HALF]
    h_ref[1] = h[:, HALF:]

    @pl.when(i == 0)
    def _():
        csx_ref[...] = jnp.zeros_like(csx_ref)
        csh_ref[...] = jnp.zeros_like(csh_ref)

    csx_ref[...] += jnp.sum(x, axis=0, keepdims=True)
    csh_ref[...] += jnp.sum(h, axis=0, keepdims=True)


def _dense1_body(agg_ref, x_ref, wn_ref, ws_ref, b_ref, cs0_ref, cs1_ref,
                 r0_ref, r1_ref, r2_ref, rb0_ref, rb1_ref, rb2_ref, out_ref):
    i = pl.program_id(0)
    agg = jnp.concatenate([agg_ref[0], agg_ref[1]], axis=1)
    x = jnp.concatenate([x_ref[0], x_ref[1]], axis=1)
    h = jnp.dot(agg, wn_ref[...], preferred_element_type=jnp.float32)
    h += jnp.dot(x, ws_ref[...], preferred_element_type=jnp.float32)
    h += b_ref[...]
    h = _lrelu(_lrelu(h))
    csh = jnp.sum(h, axis=0, keepdims=True)

    @pl.when(i == 0)
    def _():
        out_ref[...] = (
            jnp.dot(cs0_ref[...], r0_ref[...], preferred_element_type=jnp.float32)
            + jnp.dot(cs1_ref[...], r1_ref[...], preferred_element_type=jnp.float32)
            + float(N) * (rb0_ref[...] + rb1_ref[...] + rb2_ref[...])
        )

    out_ref[...] += jnp.dot(csh, r2_ref[...], preferred_element_type=jnp.float32)


GBN = 1000

_dense0 = pl.pallas_call(
    _dense0_body,
    grid=(N // GBN,),
    in_specs=[
        pl.BlockSpec((NC, GBN, HALF), lambda i: (0, i, 0)),
        pl.BlockSpec((GBN, D), lambda i: (i, 0)),
        pl.BlockSpec((D, H), lambda i: (0, 0)),
        pl.BlockSpec((D, H), lambda i: (0, 0)),
        pl.BlockSpec((1, H), lambda i: (0, 0)),
    ],
    out_specs=[
        pl.BlockSpec((NC, GBN, HALF), lambda i: (0, i, 0)),
        pl.BlockSpec((1, D), lambda i: (0, 0)),
        pl.BlockSpec((1, H), lambda i: (0, 0)),
    ],
    out_shape=[
        jax.ShapeDtypeStruct((NC, NPAD, HALF), jnp.float32),
        jax.ShapeDtypeStruct((1, D), jnp.float32),
        jax.ShapeDtypeStruct((1, H), jnp.float32),
    ],
)

_dense1 = pl.pallas_call(
    _dense1_body,
    grid=(N // GBN,),
    in_specs=[
        pl.BlockSpec((NC, GBN, HALF), lambda i: (0, i, 0)),
        pl.BlockSpec((NC, GBN, HALF), lambda i: (0, i, 0)),
        pl.BlockSpec((H, H), lambda i: (0, 0)),
        pl.BlockSpec((H, H), lambda i: (0, 0)),
        pl.BlockSpec((1, H), lambda i: (0, 0)),
        pl.BlockSpec((1, D), lambda i: (0, 0)),
        pl.BlockSpec((1, H), lambda i: (0, 0)),
        pl.BlockSpec((D, O), lambda i: (0, 0)),
        pl.BlockSpec((H, O), lambda i: (0, 0)),
        pl.BlockSpec((H, O), lambda i: (0, 0)),
        pl.BlockSpec((1, O), lambda i: (0, 0)),
        pl.BlockSpec((1, O), lambda i: (0, 0)),
        pl.BlockSpec((1, O), lambda i: (0, 0)),
    ],
    out_specs=pl.BlockSpec((1, O), lambda i: (0, 0)),
    out_shape=jax.ShapeDtypeStruct((1, O), jnp.float32),
)


def kernel(nfeats, efeats, edge_index, Wself0, Wneigh0, b0, Wself1, Wneigh1,
           b1, R0, Rb0, R1, Rb1, R2, Rb2):
    src = edge_index[0]
    dst = edge_index[1]
    src_adj = jnp.stack([src, src + NPAD]).reshape(NC, NS, NCHUNK, CH)
    dstr = dst.reshape(NS, NCHUNK, CH)
    zeros = jnp.zeros((RPT, HALF), jnp.float32)

    x0h = jnp.concatenate(
        [nfeats[:, :HALF], jnp.zeros((NPAD - N, HALF), jnp.float32),
         nfeats[:, HALF:]], axis=0)
    x0h = jnp.concatenate([x0h, jnp.zeros((NPAD - N, HALF), jnp.float32)], axis=0)
    agg0 = _seg_sum(x0h, src_adj, dstr, zeros)
    h1, cs0, cs1 = _dense0(agg0, nfeats, Wneigh0, Wself0, b0.reshape(1, H))
    agg1 = _seg_sum(h1.reshape(NC * NPAD, HALF), src_adj, dstr, zeros)
    out = _dense1(agg1, h1, Wneigh1, Wself1, b1.reshape(1, H), cs0, cs1,
                  R0, R1, R2, Rb0.reshape(1, O), Rb1.reshape(1, O),
                  Rb2.reshape(1, O))
    return out
